# baseline (speedup 1.0000x reference)
import functools

import jax
import jax.numpy as jnp
from jax import lax
from jax.experimental import pallas as pl
from jax.experimental.pallas import tpu as pltpu
from jax.experimental.pallas import tpu_sc as plsc

_NUM_CORES = 2
_NUM_SUBCORES = 16
_NUM_WORKERS = _NUM_CORES * _NUM_SUBCORES

_CHUNK = 640


@functools.lru_cache(maxsize=None)
def _build_gather(ids_shape: tuple, vocab: int, dim: int):
    num_seqs, seq_len = ids_shape
    num_ids = num_seqs * seq_len
    assert num_ids % _NUM_WORKERS == 0
    per_worker = num_ids // _NUM_WORKERS
    assert num_seqs % _NUM_WORKERS == 0
    seqs_per_worker = num_seqs // _NUM_WORKERS
    chunk = _CHUNK
    while per_worker % chunk:
        chunk //= 2
    n_chunks = per_worker // chunk
    assert n_chunks % 2 == 0 and n_chunks >= 4

    mesh = plsc.VectorSubcoreMesh(
        core_axis_name="c", subcore_axis_name="s",
        num_cores=_NUM_CORES, num_subcores=_NUM_SUBCORES,
    )

    @functools.partial(
        pl.kernel,
        out_type=jax.ShapeDtypeStruct((num_ids, 2 * dim), jnp.float32),
        mesh=mesh,
        scratch_types=[
            pltpu.VMEM((per_worker,), jnp.int32),
            pltpu.VMEM((chunk, dim), jnp.float32),
            pltpu.VMEM((chunk, dim), jnp.float32),
            pltpu.SemaphoreType.DMA,
            pltpu.SemaphoreType.DMA,
            pltpu.SemaphoreType.DMA,
            pltpu.SemaphoreType.DMA,
            pltpu.SemaphoreType.DMA,
        ],
        compiler_params=pltpu.CompilerParams(use_tc_tiling_on_sc=False),
    )
    def gather(ids_hbm, table_hbm, out_hbm, ids_v, rows0, rows1,
               isem, gsem0, gsem1, ssem0, ssem1):
        wid = lax.axis_index("s") * _NUM_CORES + lax.axis_index("c")
        base = wid * per_worker
        seq_base = wid * seqs_per_worker
        rows = (rows0, rows1)
        gsem = (gsem0, gsem1)
        ssem = (ssem0, ssem1)

        @pl.loop(0, seqs_per_worker)
        def _stage(j):
            pltpu.async_copy(ids_hbm.at[seq_base + j],
                             ids_v.at[pl.ds(j * seq_len, seq_len)], isem)

        @pl.loop(0, seqs_per_worker)
        def _drain(j):
            pltpu.make_async_copy(
                ids_hbm.at[seq_base + j],
                ids_v.at[pl.ds(j * seq_len, seq_len)], isem).wait()

        def idx_ref(i):
            return ids_v.at[pl.ds(i * chunk, chunk)]

        def start_gather(i, b):
            pltpu.async_copy(table_hbm.at[idx_ref(i)], rows[b], gsem[b])

        def wait_gather(i, b):
            pltpu.make_async_copy(
                table_hbm.at[idx_ref(i)], rows[b], gsem[b]).wait()

        def start_store(i, b):
            off = base + i * chunk
            pltpu.async_copy(
                rows[b], out_hbm.at[pl.ds(off, chunk), pl.ds(0, dim)], ssem[b])

        def wait_store(i, b):
            off = base + i * chunk
            pltpu.make_async_copy(
                rows[b], out_hbm.at[pl.ds(off, chunk), pl.ds(0, dim)],
                ssem[b]).wait()

        start_gather(0, 0)
        start_gather(1, 1)
        wait_gather(0, 0)
        start_store(0, 0)

        @pl.loop(1, n_chunks // 2)
        def _pair_loop(g):
            for b in (0, 1):
                i = 2 * g + b
                wait_store(i - 2, b)
                start_gather(i, b)
                wait_gather(i - 1, 1 - b)
                start_store(i - 1, 1 - b)

        wait_gather(n_chunks - 1, 1)
        start_store(n_chunks - 1, 1)
        wait_store(n_chunks - 2, 0)
        wait_store(n_chunks - 1, 1)

    return gather


@functools.lru_cache(maxsize=None)
def _build_detile(vocab: int, dim: int):
    bs = 4000
    while vocab % bs:
        bs //= 2
    grid = vocab // bs
    rows_out = bs * dim // 128

    @functools.partial(
        pl.pallas_call,
        grid=(grid,),
        in_specs=[pl.BlockSpec((bs, dim), lambda i: (i, 0))],
        out_specs=pl.BlockSpec((rows_out, 128), lambda i: (i, 0)),
        out_shape=jax.ShapeDtypeStruct((vocab * dim // 128, 128),
                                       jnp.float32),
    )
    def detile(x_ref, o_ref):
        x3 = x_ref[...].reshape(rows_out, 2, dim)
        o_ref[...] = jnp.concatenate([x3[:, 0, :], x3[:, 1, :]], axis=1)

    return detile


def kernel(ids, table):
    vocab, dim = table.shape
    gather = _build_gather(ids.shape, vocab, dim)
    t128 = _build_detile(vocab, dim)(table)
    out_pad = gather(ids, t128.reshape(vocab, dim))
    return out_pad[:, :dim].reshape(ids.shape + (dim,))


# speedup vs baseline: 1.2005x; 1.2005x over previous
import functools

import jax
import jax.numpy as jnp
from jax import lax
from jax.experimental import pallas as pl
from jax.experimental.pallas import tpu as pltpu
from jax.experimental.pallas import tpu_sc as plsc

_NUM_CORES = 2
_NUM_SUBCORES = 16
_NUM_WORKERS = _NUM_CORES * _NUM_SUBCORES

_CHUNK = 640


@functools.lru_cache(maxsize=None)
def _build_gather(ids_shape: tuple, vocab: int, dim: int):
    num_seqs, seq_len = ids_shape
    num_ids = num_seqs * seq_len
    assert num_ids % _NUM_WORKERS == 0
    per_worker = num_ids // _NUM_WORKERS
    assert num_seqs % _NUM_WORKERS == 0
    seqs_per_worker = num_seqs // _NUM_WORKERS
    chunk = _CHUNK
    while per_worker % chunk:
        chunk //= 2
    n_chunks = per_worker // chunk
    assert n_chunks % 2 == 0 and n_chunks >= 4

    mesh = plsc.VectorSubcoreMesh(
        core_axis_name="c", subcore_axis_name="s",
        num_cores=_NUM_CORES, num_subcores=_NUM_SUBCORES,
    )

    @functools.partial(
        pl.kernel,
        out_type=jax.ShapeDtypeStruct((num_ids, 2 * dim), jnp.float32),
        mesh=mesh,
        scratch_types=[
            pltpu.VMEM((per_worker,), jnp.int32),
            pltpu.VMEM((chunk, dim), jnp.float32),
            pltpu.VMEM((chunk, dim), jnp.float32),
            pltpu.SemaphoreType.DMA,
            pltpu.SemaphoreType.DMA,
            pltpu.SemaphoreType.DMA,
            pltpu.SemaphoreType.DMA,
            pltpu.SemaphoreType.DMA,
        ],
        compiler_params=pltpu.CompilerParams(use_tc_tiling_on_sc=False),
    )
    def gather(ids_hbm, table_hbm, out_hbm, ids_v, rows0, rows1,
               isem, gsem0, gsem1, ssem0, ssem1):
        wid = lax.axis_index("s") * _NUM_CORES + lax.axis_index("c")
        base = wid * per_worker
        seq_base = wid * seqs_per_worker
        rows = (rows0, rows1)
        gsem = (gsem0, gsem1)
        ssem = (ssem0, ssem1)

        @pl.loop(0, seqs_per_worker)
        def _stage(j):
            pltpu.async_copy(ids_hbm.at[seq_base + j],
                             ids_v.at[pl.ds(j * seq_len, seq_len)], isem)

        @pl.loop(0, seqs_per_worker)
        def _drain(j):
            pltpu.make_async_copy(
                ids_hbm.at[seq_base + j],
                ids_v.at[pl.ds(j * seq_len, seq_len)], isem).wait()

        def idx_ref(i):
            return ids_v.at[pl.ds(i * chunk, chunk)]

        def start_gather(i, b):
            pltpu.async_copy(table_hbm.at[idx_ref(i)], rows[b], gsem[b])

        def wait_gather(i, b):
            pltpu.make_async_copy(
                table_hbm.at[idx_ref(i)], rows[b], gsem[b]).wait()

        def start_store(i, b):
            off = base + i * chunk
            pltpu.async_copy(
                rows[b], out_hbm.at[pl.ds(off, chunk), pl.ds(0, dim)], ssem[b])

        def wait_store(i, b):
            off = base + i * chunk
            pltpu.make_async_copy(
                rows[b], out_hbm.at[pl.ds(off, chunk), pl.ds(0, dim)],
                ssem[b]).wait()

        start_gather(0, 0)
        start_gather(1, 1)
        wait_gather(0, 0)
        start_store(0, 0)

        @pl.loop(1, n_chunks // 2)
        def _pair_loop(g):
            for b in (0, 1):
                i = 2 * g + b
                wait_store(i - 2, b)
                start_gather(i, b)
                wait_gather(i - 1, 1 - b)
                start_store(i - 1, 1 - b)

        wait_gather(n_chunks - 1, 1)
        start_store(n_chunks - 1, 1)
        wait_store(n_chunks - 2, 0)
        wait_store(n_chunks - 1, 1)

    return gather


def kernel(ids, table):
    vocab, dim = table.shape
    gather = _build_gather(ids.shape, vocab, dim)
    out_pad = gather(ids, table)
    return out_pad[:, :dim].reshape(ids.shape + (dim,))
